# baseline (speedup 1.0000x reference)
import functools

import jax
import jax.numpy as jnp
from jax.experimental import pallas as pl
from jax.experimental.pallas import tpu as pltpu
from jax.experimental.pallas import tpu_sc as plsc


def _pick_tile(n: int) -> int:
    for t in (4000, 2048, 2000, 1600, 1280, 1024, 800, 512, 400, 256, 200, 128, 8):
        if n % t == 0 and t % 8 == 0:
            return t
    return n


def _sc_gathers(e128, w128, b128, i_src, i_tgt, i_b):
    B = i_src.shape[0]
    mesh = plsc.VectorSubcoreMesh(core_axis_name="c", subcore_axis_name="s")
    nw = 32
    bpw = B // nw

    @functools.partial(
        pl.kernel, mesh=mesh,
        out_type=(
            jax.ShapeDtypeStruct((B, 128), e128.dtype),
            jax.ShapeDtypeStruct((B, 128), w128.dtype),
            jax.ShapeDtypeStruct((B, 128), b128.dtype),
        ),
        scratch_types=[
            pltpu.VMEM((bpw,), jnp.int32),
            pltpu.VMEM((bpw,), jnp.int32),
            pltpu.VMEM((bpw,), jnp.int32),
            pltpu.VMEM((bpw, 128), jnp.float32),
            pltpu.VMEM((bpw, 128), jnp.float32),
            pltpu.VMEM((bpw, 128), jnp.float32),
            pltpu.SemaphoreType.DMA,
            pltpu.SemaphoreType.DMA,
            pltpu.SemaphoreType.DMA,
        ],
    )
    def k(e_hbm, w_hbm, b_hbm, is_hbm, it_hbm, ib_hbm,
          o1_hbm, o2_hbm, o3_hbm,
          i1_v, i2_v, i3_v, r1_v, r2_v, r3_v, s1, s2, s3):
        wid = jax.lax.axis_index("s") * 2 + jax.lax.axis_index("c")
        base = wid * bpw
        pltpu.sync_copy(is_hbm.at[pl.ds(base, bpw)], i1_v)
        pltpu.sync_copy(it_hbm.at[pl.ds(base, bpw)], i2_v)
        pltpu.sync_copy(ib_hbm.at[pl.ds(base, bpw)], i3_v)
        c1 = pltpu.async_copy(e_hbm.at[i1_v], r1_v, s1)
        c2 = pltpu.async_copy(w_hbm.at[i2_v], r2_v, s2)
        c3 = pltpu.async_copy(b_hbm.at[i3_v], r3_v, s3)
        c1.wait()
        c2.wait()
        c3.wait()
        pltpu.sync_copy(r1_v, o1_hbm.at[pl.ds(base, bpw)])
        pltpu.sync_copy(r2_v, o2_hbm.at[pl.ds(base, bpw)])
        pltpu.sync_copy(r3_v, o3_hbm.at[pl.ds(base, bpw)])

    return k(e128, w128, b128, i_src, i_tgt, i_b)


def _tc_body(g2_ref, src_ref, wh_ref, bh_ref, wo_ref, bo_ref,
             w2_ref, tgt_ref, b2_ref,
             lat_out, loss_out, lat_sc, acc_sc, *, nt, d, tile):
    j = pl.program_id(0)

    @pl.when(j == 0)
    def _init():
        g2 = g2_ref[...]
        smod = jnp.bitwise_and(src_ref[...], 1)
        gsel = jnp.where(smod == 0, g2[:, :d], g2[:, d:])
        lat = jax.lax.dot_general(
            gsel, wh_ref[...],
            (((1,), (1,)), ((), ())), preferred_element_type=jnp.float32)
        lat = lat + bh_ref[...]
        nrm = jnp.sqrt(jnp.sum(lat * lat, axis=1, keepdims=True))
        den = jnp.where(nrm == 0.0, 1.0, nrm)
        lat = lat / den
        lat_sc[...] = lat
        lat_out[...] = lat
        acc_sc[...] = jnp.zeros_like(acc_sc)

    lat = lat_sc[...]
    lat_bf = lat.astype(jnp.bfloat16)
    bo_bf = bo_ref[0].astype(jnp.bfloat16)
    sub = tile // 4
    nfull = sub // 128
    rem = sub - nfull * 128
    for c in range(4):
        wsub = wo_ref[c * sub:(c + 1) * sub, :].astype(jnp.bfloat16)
        lg = jax.lax.dot_general(
            lat_bf, wsub,
            (((1,), (1,)), ((), ())), preferred_element_type=jnp.float32)
        ex = jnp.exp(lg.astype(jnp.bfloat16) + bo_bf[:, c * sub:(c + 1) * sub])
        s = ex[:, :128]
        for k in range(1, nfull):
            s = s + ex[:, k * 128:(k + 1) * 128]
        if rem:
            lanes = jax.lax.broadcasted_iota(jnp.int32, ex[:, :128].shape, 1)
            s = s + jnp.where(lanes < 128 - rem, jnp.bfloat16(0.0),
                              ex[:, sub - 128:])
        acc_sc[...] += s.astype(jnp.float32)

    @pl.when(j == nt - 1)
    def _fin():
        b = lat.shape[0]
        w2 = w2_ref[...]
        tmod = jnp.bitwise_and(tgt_ref[...], 1)
        wsel = jnp.where(tmod == 0, w2[:, :d], w2[:, d:])
        tgt_dot = jnp.sum(lat * wsel, axis=1)
        onehot = (jax.lax.broadcasted_iota(jnp.int32, (b, 128), 1)
                  == jnp.bitwise_and(tgt_ref[...], 127))
        tgt_b = jnp.sum(jnp.where(onehot, b2_ref[...], 0.0), axis=1)
        lse = jnp.log(jnp.sum(acc_sc[...], axis=1))
        loss_out[...] = jnp.mean(lse - tgt_dot - tgt_b).reshape(1, 1)


def kernel(source, targets, emb, W_h, b_h, W_out, b_out):
    B = source.shape[0]
    N, D = emb.shape
    L = W_h.shape[0]
    src = source.astype(jnp.int32)
    tgt = targets.astype(jnp.int32)

    e128 = jnp.concatenate([emb[0::2], emb[1::2]], axis=1)
    w128 = jnp.concatenate([W_out[0::2], W_out[1::2]], axis=1)
    npad = (-N) % 128
    b128 = jnp.pad(b_out, (0, npad)).reshape((N + npad) // 128, 128)

    g2, w2, b2 = _sc_gathers(e128, w128, b128, src // 2, tgt // 2, tgt // 128)

    tile = _pick_tile(N)
    nt = N // tile
    b3 = b_out.reshape(nt, 1, tile)

    grid_spec = pltpu.PrefetchScalarGridSpec(
        num_scalar_prefetch=0,
        grid=(nt,),
        in_specs=[
            pl.BlockSpec((B, 2 * D), lambda j: (0, 0)),
            pl.BlockSpec((B, 1), lambda j: (0, 0)),
            pl.BlockSpec((L, D), lambda j: (0, 0)),
            pl.BlockSpec((1, L), lambda j: (0, 0)),
            pl.BlockSpec((tile, L), lambda j: (j, 0)),
            pl.BlockSpec((1, 1, tile), lambda j: (j, 0, 0)),
            pl.BlockSpec((B, 2 * L), lambda j: (0, 0)),
            pl.BlockSpec((B, 1), lambda j: (0, 0)),
            pl.BlockSpec((B, 128), lambda j: (0, 0)),
        ],
        out_specs=[
            pl.BlockSpec((B, L), lambda j: (0, 0)),
            pl.BlockSpec((1, 1), lambda j: (0, 0)),
        ],
        scratch_shapes=[
            pltpu.VMEM((B, L), jnp.float32),
            pltpu.VMEM((B, 128), jnp.float32),
        ],
    )

    latent, loss = pl.pallas_call(
        functools.partial(_tc_body, nt=nt, d=D, tile=tile),
        grid_spec=grid_spec,
        out_shape=[
            jax.ShapeDtypeStruct((B, L), jnp.float32),
            jax.ShapeDtypeStruct((1, 1), jnp.float32),
        ],
        compiler_params=pltpu.CompilerParams(
            dimension_semantics=("arbitrary",),
        ),
    )(g2, src.reshape(B, 1), W_h, b_h.reshape(1, L), W_out, b3, w2,
      tgt.reshape(B, 1), b2)

    return latent, loss.reshape(())


# speedup vs baseline: 6.0594x; 6.0594x over previous
import functools

import jax
import jax.numpy as jnp
from jax.experimental import pallas as pl
from jax.experimental.pallas import tpu as pltpu
from jax.experimental.pallas import tpu_sc as plsc


def _pick_tile(n: int) -> int:
    for t in (4000, 2048, 2000, 1600, 1280, 1024, 800, 512, 400, 256, 200, 128, 8):
        if n % t == 0 and t % 8 == 0:
            return t
    return n


def _sc_gathers(emb, w_out, b64, i_src, i_tgt, i_b):
    B = i_src.shape[0]
    D = emb.shape[1]
    mesh = plsc.ScalarSubcoreMesh(axis_name="core", num_cores=2)
    hpc = B // 2

    @functools.partial(
        pl.kernel, mesh=mesh,
        out_type=(
            jax.ShapeDtypeStruct((B, D), emb.dtype),
            jax.ShapeDtypeStruct((B, D), w_out.dtype),
            jax.ShapeDtypeStruct((B, D), b64.dtype),
        ),
        scratch_types=[
            pltpu.SMEM((hpc,), jnp.int32),
            pltpu.SMEM((hpc,), jnp.int32),
            pltpu.SMEM((hpc,), jnp.int32),
            pltpu.SemaphoreType.DMA,
            pltpu.SemaphoreType.DMA,
            pltpu.SemaphoreType.DMA,
            pltpu.SemaphoreType.DMA,
        ],
    )
    def k(e_hbm, w_hbm, b_hbm, is_hbm, it_hbm, ib_hbm,
          o1_hbm, o2_hbm, o3_hbm,
          i1_s, i2_s, i3_s, s0, s1, s2, s3):
        cid = jax.lax.axis_index("core")
        base = cid * hpc
        pltpu.async_copy(is_hbm.at[pl.ds(base, hpc)], i1_s, s0).wait()
        pltpu.async_copy(it_hbm.at[pl.ds(base, hpc)], i2_s, s0).wait()
        pltpu.async_copy(ib_hbm.at[pl.ds(base, hpc)], i3_s, s0).wait()

        @pl.loop(0, hpc)
        def _issue(i):
            pltpu.async_copy(e_hbm.at[i1_s[i]], o1_hbm.at[base + i], s1)
            pltpu.async_copy(w_hbm.at[i2_s[i]], o2_hbm.at[base + i], s2)
            pltpu.async_copy(b_hbm.at[i3_s[i]], o3_hbm.at[base + i], s3)

        @pl.loop(0, hpc)
        def _drain(i):
            pltpu.make_async_copy(e_hbm.at[0], o1_hbm.at[0], s1).wait()
            pltpu.make_async_copy(w_hbm.at[0], o2_hbm.at[0], s2).wait()
            pltpu.make_async_copy(b_hbm.at[0], o3_hbm.at[0], s3).wait()

    return k(emb, w_out, b64, i_src, i_tgt, i_b)


def _tc_body(g_ref, wh_ref, bh_ref, wo_ref, bo_ref,
             w2_ref, tgt_ref, b2_ref,
             lat_out, loss_out, lat_sc, acc_sc, *, nt, d, tile):
    j = pl.program_id(0)

    @pl.when(j == 0)
    def _init():
        gsel = g_ref[...]
        lat = jax.lax.dot_general(
            gsel, wh_ref[...],
            (((1,), (1,)), ((), ())), preferred_element_type=jnp.float32)
        lat = lat + bh_ref[...]
        nrm = jnp.sqrt(jnp.sum(lat * lat, axis=1, keepdims=True))
        den = jnp.where(nrm == 0.0, 1.0, nrm)
        lat = lat / den
        lat_sc[...] = lat
        lat_out[...] = lat
        acc_sc[...] = jnp.zeros_like(acc_sc)

    lat = lat_sc[...]
    lat_bf = lat.astype(jnp.bfloat16)
    bo_bf = bo_ref[0].astype(jnp.bfloat16)
    sub = tile // 4
    nfull = sub // 128
    rem = sub - nfull * 128
    for c in range(4):
        wsub = wo_ref[c * sub:(c + 1) * sub, :].astype(jnp.bfloat16)
        lg = jax.lax.dot_general(
            lat_bf, wsub,
            (((1,), (1,)), ((), ())), preferred_element_type=jnp.float32)
        ex = jnp.exp(lg.astype(jnp.bfloat16) + bo_bf[:, c * sub:(c + 1) * sub])
        s = ex[:, :128]
        for k in range(1, nfull):
            s = s + ex[:, k * 128:(k + 1) * 128]
        if rem:
            lanes = jax.lax.broadcasted_iota(jnp.int32, ex[:, :128].shape, 1)
            s = s + jnp.where(lanes < 128 - rem, jnp.bfloat16(0.0),
                              ex[:, sub - 128:])
        acc_sc[...] += s.astype(jnp.float32)

    @pl.when(j == nt - 1)
    def _fin():
        b = lat.shape[0]
        tgt_dot = jnp.sum(lat * w2_ref[...], axis=1)
        onehot = (jax.lax.broadcasted_iota(jnp.int32, (b, 64), 1)
                  == jnp.bitwise_and(tgt_ref[...], 63))
        tgt_b = jnp.sum(jnp.where(onehot, b2_ref[...], 0.0), axis=1)
        lse = jnp.log(jnp.sum(acc_sc[...], axis=1))
        loss_out[...] = jnp.mean(lse - tgt_dot - tgt_b).reshape(1, 1)


def kernel(source, targets, emb, W_h, b_h, W_out, b_out):
    B = source.shape[0]
    N, D = emb.shape
    L = W_h.shape[0]
    src = source.astype(jnp.int32)
    tgt = targets.astype(jnp.int32)

    npad = (-N) % 64
    b64 = jnp.pad(b_out, (0, npad)).reshape((N + npad) // 64, 64)

    g2, w2, b2 = _sc_gathers(emb, W_out, b64, src, tgt, tgt // 64)

    tile = _pick_tile(N)
    nt = N // tile
    b3 = b_out.reshape(nt, 1, tile)

    grid_spec = pltpu.PrefetchScalarGridSpec(
        num_scalar_prefetch=0,
        grid=(nt,),
        in_specs=[
            pl.BlockSpec((B, D), lambda j: (0, 0)),
            pl.BlockSpec((L, D), lambda j: (0, 0)),
            pl.BlockSpec((1, L), lambda j: (0, 0)),
            pl.BlockSpec((tile, L), lambda j: (j, 0)),
            pl.BlockSpec((1, 1, tile), lambda j: (j, 0, 0)),
            pl.BlockSpec((B, L), lambda j: (0, 0)),
            pl.BlockSpec((B, 1), lambda j: (0, 0)),
            pl.BlockSpec((B, D), lambda j: (0, 0)),
        ],
        out_specs=[
            pl.BlockSpec((B, L), lambda j: (0, 0)),
            pl.BlockSpec((1, 1), lambda j: (0, 0)),
        ],
        scratch_shapes=[
            pltpu.VMEM((B, L), jnp.float32),
            pltpu.VMEM((B, 128), jnp.float32),
        ],
    )

    latent, loss = pl.pallas_call(
        functools.partial(_tc_body, nt=nt, d=D, tile=tile),
        grid_spec=grid_spec,
        out_shape=[
            jax.ShapeDtypeStruct((B, L), jnp.float32),
            jax.ShapeDtypeStruct((1, 1), jnp.float32),
        ],
        compiler_params=pltpu.CompilerParams(
            dimension_semantics=("arbitrary",),
        ),
    )(g2, W_h, b_h.reshape(1, L), W_out, b3, w2, tgt.reshape(B, 1), b2)

    return latent, loss.reshape(())


# speedup vs baseline: 6.1601x; 1.0166x over previous
import functools

import jax
import jax.numpy as jnp
from jax.experimental import pallas as pl
from jax.experimental.pallas import tpu as pltpu
from jax.experimental.pallas import tpu_sc as plsc


def _pick_tile(n: int) -> int:
    for t in (5000, 4000, 2048, 2000, 1600, 1280, 1024, 800, 512, 400, 256, 200, 128, 8):
        if n % t == 0 and t % 8 == 0:
            return t
    return n


def _sc_gathers(emb, w_out, b64, i_src, i_tgt, i_b):
    B = i_src.shape[0]
    D = emb.shape[1]
    mesh = plsc.ScalarSubcoreMesh(axis_name="core", num_cores=2)
    hpc = B // 2

    @functools.partial(
        pl.kernel, mesh=mesh,
        out_type=(
            jax.ShapeDtypeStruct((B, D), emb.dtype),
            jax.ShapeDtypeStruct((B, D), w_out.dtype),
            jax.ShapeDtypeStruct((B, D), b64.dtype),
        ),
        scratch_types=[
            pltpu.SMEM((hpc,), jnp.int32),
            pltpu.SMEM((hpc,), jnp.int32),
            pltpu.SMEM((hpc,), jnp.int32),
            pltpu.SemaphoreType.DMA,
            pltpu.SemaphoreType.DMA,
            pltpu.SemaphoreType.DMA,
            pltpu.SemaphoreType.DMA,
        ],
    )
    def k(e_hbm, w_hbm, b_hbm, is_hbm, it_hbm, ib_hbm,
          o1_hbm, o2_hbm, o3_hbm,
          i1_s, i2_s, i3_s, s0, s1, s2, s3):
        cid = jax.lax.axis_index("core")
        base = cid * hpc
        pltpu.async_copy(is_hbm.at[pl.ds(base, hpc)], i1_s, s0).wait()
        pltpu.async_copy(it_hbm.at[pl.ds(base, hpc)], i2_s, s0).wait()
        pltpu.async_copy(ib_hbm.at[pl.ds(base, hpc)], i3_s, s0).wait()

        @pl.loop(0, hpc)
        def _issue(i):
            pltpu.async_copy(e_hbm.at[i1_s[i]], o1_hbm.at[base + i], s1)
            pltpu.async_copy(w_hbm.at[i2_s[i]], o2_hbm.at[base + i], s2)
            pltpu.async_copy(b_hbm.at[i3_s[i]], o3_hbm.at[base + i], s3)

        pltpu.make_async_copy(
            o1_hbm.at[pl.ds(base, hpc)], o1_hbm.at[pl.ds(base, hpc)], s1).wait()
        pltpu.make_async_copy(
            o2_hbm.at[pl.ds(base, hpc)], o2_hbm.at[pl.ds(base, hpc)], s2).wait()
        pltpu.make_async_copy(
            o3_hbm.at[pl.ds(base, hpc)], o3_hbm.at[pl.ds(base, hpc)], s3).wait()

    return k(emb, w_out, b64, i_src, i_tgt, i_b)


def _tc_body(g_ref, wh_ref, bh_ref, wo_ref, bo_ref,
             w2_ref, tgt_ref, b2_ref,
             lat_out, loss_out, lat_sc, acc_sc, *, nt, d, tile):
    j = pl.program_id(0)

    @pl.when(j == 0)
    def _init():
        gsel = g_ref[...]
        lat = jax.lax.dot_general(
            gsel, wh_ref[...],
            (((1,), (1,)), ((), ())), preferred_element_type=jnp.float32)
        lat = lat + bh_ref[...]
        nrm = jnp.sqrt(jnp.sum(lat * lat, axis=1, keepdims=True))
        den = jnp.where(nrm == 0.0, 1.0, nrm)
        lat = lat / den
        lat_sc[...] = lat
        lat_out[...] = lat
        acc_sc[...] = jnp.zeros_like(acc_sc)

    lat = lat_sc[...]
    lat_bf = lat.astype(jnp.bfloat16)
    bo_bf = bo_ref[0].astype(jnp.bfloat16)
    sub = tile // 4
    nfull = sub // 128
    rem = sub - nfull * 128
    for c in range(4):
        wsub = wo_ref[c * sub:(c + 1) * sub, :].astype(jnp.bfloat16)
        lg = jax.lax.dot_general(
            lat_bf, wsub,
            (((1,), (1,)), ((), ())), preferred_element_type=jnp.float32)
        ex = jnp.exp(lg.astype(jnp.bfloat16) + bo_bf[:, c * sub:(c + 1) * sub])
        s = ex[:, :128]
        for k in range(1, nfull):
            s = s + ex[:, k * 128:(k + 1) * 128]
        if rem:
            lanes = jax.lax.broadcasted_iota(jnp.int32, ex[:, :128].shape, 1)
            s = s + jnp.where(lanes < 128 - rem, jnp.bfloat16(0.0),
                              ex[:, sub - 128:])
        acc_sc[...] += s.astype(jnp.float32)

    @pl.when(j == nt - 1)
    def _fin():
        b = lat.shape[0]
        tgt_dot = jnp.sum(lat * w2_ref[...], axis=1)
        onehot = (jax.lax.broadcasted_iota(jnp.int32, (b, 64), 1)
                  == jnp.bitwise_and(tgt_ref[...], 63))
        tgt_b = jnp.sum(jnp.where(onehot, b2_ref[...], 0.0), axis=1)
        lse = jnp.log(jnp.sum(acc_sc[...], axis=1))
        loss_out[...] = jnp.mean(lse - tgt_dot - tgt_b).reshape(1, 1)


def kernel(source, targets, emb, W_h, b_h, W_out, b_out):
    B = source.shape[0]
    N, D = emb.shape
    L = W_h.shape[0]
    src = source.astype(jnp.int32)
    tgt = targets.astype(jnp.int32)

    npad = (-N) % 64
    b64 = jnp.pad(b_out, (0, npad)).reshape((N + npad) // 64, 64)

    g2, w2, b2 = _sc_gathers(emb, W_out, b64, src, tgt, tgt // 64)

    tile = _pick_tile(N)
    nt = N // tile
    b3 = b_out.reshape(nt, 1, tile)

    grid_spec = pltpu.PrefetchScalarGridSpec(
        num_scalar_prefetch=0,
        grid=(nt,),
        in_specs=[
            pl.BlockSpec((B, D), lambda j: (0, 0)),
            pl.BlockSpec((L, D), lambda j: (0, 0)),
            pl.BlockSpec((1, L), lambda j: (0, 0)),
            pl.BlockSpec((tile, L), lambda j: (j, 0)),
            pl.BlockSpec((1, 1, tile), lambda j: (j, 0, 0)),
            pl.BlockSpec((B, L), lambda j: (0, 0)),
            pl.BlockSpec((B, 1), lambda j: (0, 0)),
            pl.BlockSpec((B, D), lambda j: (0, 0)),
        ],
        out_specs=[
            pl.BlockSpec((B, L), lambda j: (0, 0)),
            pl.BlockSpec((1, 1), lambda j: (0, 0)),
        ],
        scratch_shapes=[
            pltpu.VMEM((B, L), jnp.float32),
            pltpu.VMEM((B, 128), jnp.float32),
        ],
    )

    latent, loss = pl.pallas_call(
        functools.partial(_tc_body, nt=nt, d=D, tile=tile),
        grid_spec=grid_spec,
        out_shape=[
            jax.ShapeDtypeStruct((B, L), jnp.float32),
            jax.ShapeDtypeStruct((1, 1), jnp.float32),
        ],
        compiler_params=pltpu.CompilerParams(
            dimension_semantics=("arbitrary",),
        ),
    )(g2, W_h, b_h.reshape(1, L), W_out, b3, w2, tgt.reshape(B, 1), b2)

    return latent, loss.reshape(())


# speedup vs baseline: 7.9467x; 1.2900x over previous
import functools

import jax
import jax.numpy as jnp
from jax.experimental import pallas as pl
from jax.experimental.pallas import tpu as pltpu
from jax.experimental.pallas import tpu_sc as plsc


def _pick_tile(n: int) -> int:
    for t in (5000, 4000, 2048, 2000, 1600, 1280, 1024, 800, 512, 400, 256,
              200, 128, 8):
        if n % t == 0 and t % 8 == 0:
            return t
    return n


def _sc_row_gather(tables, idxs):
    B = idxs[0].shape[0]
    n = len(tables)
    mesh = plsc.ScalarSubcoreMesh(axis_name="core", num_cores=2)
    hpc = B // 2

    @functools.partial(
        pl.kernel, mesh=mesh,
        out_type=tuple(
            jax.ShapeDtypeStruct((B, t.shape[1]), t.dtype) for t in tables),
        scratch_types=(
            [pltpu.SMEM((hpc,), jnp.int32) for _ in range(n)]
            + [pltpu.SemaphoreType.DMA for _ in range(n + 1)]),
    )
    def k(*refs):
        t_hbm = refs[:n]
        i_hbm = refs[n:2 * n]
        o_hbm = refs[2 * n:3 * n]
        i_s = refs[3 * n:4 * n]
        s0 = refs[4 * n]
        sems = refs[4 * n + 1:]
        cid = jax.lax.axis_index("core")
        base = cid * hpc
        for t in range(n):
            pltpu.async_copy(i_hbm[t].at[pl.ds(base, hpc)], i_s[t], s0).wait()

        @pl.loop(0, hpc)
        def _issue(i):
            for t in range(n):
                pltpu.async_copy(
                    t_hbm[t].at[i_s[t][i]], o_hbm[t].at[base + i], sems[t])

        for t in range(n):
            pltpu.make_async_copy(
                o_hbm[t].at[pl.ds(base, hpc)],
                o_hbm[t].at[pl.ds(base, hpc)], sems[t]).wait()

    return k(*tables, *idxs)


def _tc_stream_body(g_ref, wh_ref, bh_ref, wo_ref, bo_ref,
                    lat_out, acc_out, lat_sc, acc_sc, *, nt, tile):
    j = pl.program_id(0)

    @pl.when(j == 0)
    def _init():
        lat = jax.lax.dot_general(
            g_ref[...], wh_ref[...],
            (((1,), (1,)), ((), ())), preferred_element_type=jnp.float32)
        lat = lat + bh_ref[...]
        nrm = jnp.sqrt(jnp.sum(lat * lat, axis=1, keepdims=True))
        den = jnp.where(nrm == 0.0, 1.0, nrm)
        lat = lat / den
        lat_sc[...] = lat
        lat_out[...] = lat
        acc_sc[...] = jnp.zeros_like(acc_sc)

    lat_bf = lat_sc[...].astype(jnp.bfloat16)
    bo_bf = bo_ref[0].astype(jnp.bfloat16)
    sub = tile // 4
    nfull = sub // 128
    rem = sub - nfull * 128
    for c in range(4):
        wsub = wo_ref[c * sub:(c + 1) * sub, :].astype(jnp.bfloat16)
        lg = jax.lax.dot_general(
            lat_bf, wsub,
            (((1,), (1,)), ((), ())), preferred_element_type=jnp.float32)
        ex = jnp.exp(lg.astype(jnp.bfloat16) + bo_bf[:, c * sub:(c + 1) * sub])
        s = ex[:, :128]
        for k in range(1, nfull):
            s = s + ex[:, k * 128:(k + 1) * 128]
        if rem:
            lanes = jax.lax.broadcasted_iota(jnp.int32, ex[:, :128].shape, 1)
            s = s + jnp.where(lanes < 128 - rem, jnp.bfloat16(0.0),
                              ex[:, sub - 128:])
        acc_sc[...] += s.astype(jnp.float32)

    @pl.when(j == nt - 1)
    def _fin():
        acc_out[...] = acc_sc[...]


def _tc_finish_body(lat_ref, acc_ref, w2_ref, b2_ref, tgt_ref, loss_out):
    b = lat_ref.shape[0]
    tgt_dot = jnp.sum(lat_ref[...] * w2_ref[...], axis=1)
    onehot = (jax.lax.broadcasted_iota(jnp.int32, (b, 64), 1)
              == jnp.bitwise_and(tgt_ref[...], 63))
    tgt_b = jnp.sum(jnp.where(onehot, b2_ref[...], 0.0), axis=1)
    lse = jnp.log(jnp.sum(acc_ref[...], axis=1))
    loss_out[...] = jnp.mean(lse - tgt_dot - tgt_b).reshape(1, 1)


def kernel(source, targets, emb, W_h, b_h, W_out, b_out):
    B = source.shape[0]
    N, D = emb.shape
    L = W_h.shape[0]
    src = source.astype(jnp.int32)
    tgt = targets.astype(jnp.int32)

    npad = (-N) % 64
    b64 = jnp.pad(b_out, (0, npad)).reshape((N + npad) // 64, 64)

    (g2,) = _sc_row_gather((emb,), (src,))
    w2, b2 = _sc_row_gather((W_out, b64), (tgt, tgt // 64))

    tile = _pick_tile(N)
    nt = N // tile
    b3 = b_out.reshape(nt, 1, tile)

    grid_spec = pltpu.PrefetchScalarGridSpec(
        num_scalar_prefetch=0,
        grid=(nt,),
        in_specs=[
            pl.BlockSpec((B, D), lambda j: (0, 0)),
            pl.BlockSpec((L, D), lambda j: (0, 0)),
            pl.BlockSpec((1, L), lambda j: (0, 0)),
            pl.BlockSpec((tile, L), lambda j: (j, 0)),
            pl.BlockSpec((1, 1, tile), lambda j: (j, 0, 0)),
        ],
        out_specs=[
            pl.BlockSpec((B, L), lambda j: (0, 0)),
            pl.BlockSpec((B, 128), lambda j: (0, 0)),
        ],
        scratch_shapes=[
            pltpu.VMEM((B, L), jnp.float32),
            pltpu.VMEM((B, 128), jnp.float32),
        ],
    )

    latent, acc = pl.pallas_call(
        functools.partial(_tc_stream_body, nt=nt, tile=tile),
        grid_spec=grid_spec,
        out_shape=[
            jax.ShapeDtypeStruct((B, L), jnp.float32),
            jax.ShapeDtypeStruct((B, 128), jnp.float32),
        ],
        compiler_params=pltpu.CompilerParams(
            dimension_semantics=("arbitrary",),
        ),
    )(g2, W_h, b_h.reshape(1, L), W_out, b3)

    loss = pl.pallas_call(
        _tc_finish_body,
        out_shape=jax.ShapeDtypeStruct((1, 1), jnp.float32),
    )(latent, acc, w2, b2, tgt.reshape(B, 1))

    return latent, loss.reshape(())


# speedup vs baseline: 8.0211x; 1.0094x over previous
import functools

import jax
import jax.numpy as jnp
from jax.experimental import pallas as pl
from jax.experimental.pallas import tpu as pltpu
from jax.experimental.pallas import tpu_sc as plsc


def _pick_tile(n: int) -> int:
    for t in (10000, 5000, 4000, 2048, 2000, 1600, 1280, 1024, 800, 512, 400,
              256, 200, 128, 8):
        if n % t == 0 and t % 8 == 0:
            return t
    return n


def _sc_row_gather(tables, idxs):
    B = idxs[0].shape[0]
    n = len(tables)
    mesh = plsc.ScalarSubcoreMesh(axis_name="core", num_cores=2)
    hpc = B // 2

    @functools.partial(
        pl.kernel, mesh=mesh,
        out_type=tuple(
            jax.ShapeDtypeStruct((B, t.shape[1]), t.dtype) for t in tables),
        scratch_types=(
            [pltpu.SMEM((hpc,), jnp.int32) for _ in range(n)]
            + [pltpu.SemaphoreType.DMA for _ in range(n + 1)]),
    )
    def k(*refs):
        t_hbm = refs[:n]
        i_hbm = refs[n:2 * n]
        o_hbm = refs[2 * n:3 * n]
        i_s = refs[3 * n:4 * n]
        s0 = refs[4 * n]
        sems = refs[4 * n + 1:]
        cid = jax.lax.axis_index("core")
        base = cid * hpc
        for t in range(n):
            pltpu.async_copy(i_hbm[t].at[pl.ds(base, hpc)], i_s[t], s0).wait()

        @pl.loop(0, hpc)
        def _issue(i):
            for t in range(n):
                pltpu.async_copy(
                    t_hbm[t].at[i_s[t][i]], o_hbm[t].at[base + i], sems[t])

        for t in range(n):
            pltpu.make_async_copy(
                o_hbm[t].at[pl.ds(base, hpc)],
                o_hbm[t].at[pl.ds(base, hpc)], sems[t]).wait()

    return k(*tables, *idxs)


def _tc_stream_body(g_ref, wh_ref, bh_ref, wo_ref, bo_ref,
                    lat_out, acc_out, lat_sc, acc_sc, *, nt, tile):
    j = pl.program_id(0)

    @pl.when(j == 0)
    def _init():
        lat = jax.lax.dot_general(
            g_ref[...], wh_ref[...],
            (((1,), (1,)), ((), ())), preferred_element_type=jnp.float32)
        lat = lat + bh_ref[...]
        nrm = jnp.sqrt(jnp.sum(lat * lat, axis=1, keepdims=True))
        den = jnp.where(nrm == 0.0, 1.0, nrm)
        lat = lat / den
        lat_sc[...] = lat
        lat_out[...] = lat
        acc_sc[...] = jnp.zeros_like(acc_sc)

    lat_bf = lat_sc[...].astype(jnp.bfloat16)
    bo_bf = bo_ref[0].astype(jnp.bfloat16)
    sub = tile // 4
    nfull = sub // 128
    rem = sub - nfull * 128
    for c in range(4):
        wsub = wo_ref[c * sub:(c + 1) * sub, :].astype(jnp.bfloat16)
        lg = jax.lax.dot_general(
            lat_bf, wsub,
            (((1,), (1,)), ((), ())), preferred_element_type=jnp.float32)
        ex = jnp.exp(lg.astype(jnp.bfloat16) + bo_bf[:, c * sub:(c + 1) * sub])
        s = ex[:, :128]
        for k in range(1, nfull):
            s = s + ex[:, k * 128:(k + 1) * 128]
        if rem:
            lanes = jax.lax.broadcasted_iota(jnp.int32, ex[:, :128].shape, 1)
            s = s + jnp.where(lanes < 128 - rem, jnp.bfloat16(0.0),
                              ex[:, sub - 128:])
        acc_sc[...] += s.astype(jnp.float32)

    @pl.when(j == nt - 1)
    def _fin():
        acc_out[...] = acc_sc[...]


def _tc_finish_body(lat_ref, acc_ref, w2_ref, b2_ref, tgt_ref, loss_out):
    b = lat_ref.shape[0]
    tgt_dot = jnp.sum(lat_ref[...] * w2_ref[...], axis=1)
    onehot = (jax.lax.broadcasted_iota(jnp.int32, (b, 64), 1)
              == jnp.bitwise_and(tgt_ref[...], 63))
    tgt_b = jnp.sum(jnp.where(onehot, b2_ref[...], 0.0), axis=1)
    lse = jnp.log(jnp.sum(acc_ref[...], axis=1))
    loss_out[...] = jnp.mean(lse - tgt_dot - tgt_b).reshape(1, 1)


def kernel(source, targets, emb, W_h, b_h, W_out, b_out):
    B = source.shape[0]
    N, D = emb.shape
    L = W_h.shape[0]
    src = source.astype(jnp.int32)
    tgt = targets.astype(jnp.int32)

    npad = (-N) % 64
    b64 = jnp.pad(b_out, (0, npad)).reshape((N + npad) // 64, 64)

    (g2,) = _sc_row_gather((emb,), (src,))
    w2, b2 = _sc_row_gather((W_out, b64), (tgt, tgt // 64))

    tile = _pick_tile(N)
    nt = N // tile
    b3 = b_out.reshape(nt, 1, tile)

    grid_spec = pltpu.PrefetchScalarGridSpec(
        num_scalar_prefetch=0,
        grid=(nt,),
        in_specs=[
            pl.BlockSpec((B, D), lambda j: (0, 0)),
            pl.BlockSpec((L, D), lambda j: (0, 0)),
            pl.BlockSpec((1, L), lambda j: (0, 0)),
            pl.BlockSpec((tile, L), lambda j: (j, 0)),
            pl.BlockSpec((1, 1, tile), lambda j: (j, 0, 0)),
        ],
        out_specs=[
            pl.BlockSpec((B, L), lambda j: (0, 0)),
            pl.BlockSpec((B, 128), lambda j: (0, 0)),
        ],
        scratch_shapes=[
            pltpu.VMEM((B, L), jnp.float32),
            pltpu.VMEM((B, 128), jnp.float32),
        ],
    )

    latent, acc = pl.pallas_call(
        functools.partial(_tc_stream_body, nt=nt, tile=tile),
        grid_spec=grid_spec,
        out_shape=[
            jax.ShapeDtypeStruct((B, L), jnp.float32),
            jax.ShapeDtypeStruct((B, 128), jnp.float32),
        ],
        compiler_params=pltpu.CompilerParams(
            dimension_semantics=("arbitrary",),
        ),
    )(g2, W_h, b_h.reshape(1, L), W_out, b3)

    loss = pl.pallas_call(
        _tc_finish_body,
        out_shape=jax.ShapeDtypeStruct((1, 1), jnp.float32),
    )(latent, acc, w2, b2, tgt.reshape(B, 1))

    return latent, loss.reshape(())
